# Initial kernel scaffold; baseline (speedup 1.0000x reference)
import jax
import jax.numpy as jnp
from jax.experimental import pallas as pl


def kernel(x):
    raise NotImplementedError("write your pallas kernel here")


# speedup vs baseline: 1.5800x; 1.5800x over previous
import functools

import jax
import jax.numpy as jnp
from jax.experimental import pallas as pl
from jax.experimental.pallas import tpu as pltpu

N = 4096
K = N // 2
LOGN = 12
ROWS = 256


def _cmp_exchange(x, j, k, d):
    pu = jnp.roll(x, d, axis=1)
    pd = jnp.roll(x, -d, axis=1)
    lower = (j & d) == 0
    desc = (j & k) == 0
    keep_max = desc != jnp.logical_not(lower)
    partner = jnp.where(lower, pd, pu)
    return jnp.where(keep_max, jnp.maximum(x, partner),
                     jnp.minimum(x, partner))


def _sort_body(x_ref, o_ref):
    x = x_ref[...]
    j = jax.lax.broadcasted_iota(jnp.int32, (ROWS, N), 1)
    for logk in range(1, LOGN):
        k = 1 << logk
        for logd in range(logk - 1, -1, -1):
            x = _cmp_exchange(x, j, k, 1 << logd)
    x = _cmp_exchange(x, j, N, N // 2)
    x = x[:, :K]
    jtop = j[:, :K]
    for logd in range(LOGN - 2, -1, -1):
        x = _cmp_exchange(x, jtop, N, 1 << logd)
    o_ref[...] = x


@jax.jit
def kernel(x):
    b, t, n = x.shape
    rows = b * t
    flat = x.reshape(rows, n)
    out = pl.pallas_call(
        _sort_body,
        grid=(rows // ROWS,),
        in_specs=[pl.BlockSpec((ROWS, N), lambda i: (i, 0))],
        out_specs=pl.BlockSpec((ROWS, K), lambda i: (i, 0)),
        out_shape=jax.ShapeDtypeStruct((rows, K), jnp.float32),
        compiler_params=pltpu.CompilerParams(
            dimension_semantics=("arbitrary",),
        ),
    )(flat)
    return out.reshape(b, t, K)
